# baseline (speedup 1.0000x reference)
import jax
import jax.numpy as jnp
from jax import lax
from jax.experimental import pallas as pl
from jax.experimental.pallas import tpu as pltpu

N_IN = 162
N_OUT = 42
NEIGH = 7
U = 4
NC = N_OUT * NEIGH
CB = 512
NCB = 2048 // CB


def _body(ck_ref, cv_ref, cw_ref, x_ref, o_ref):
    s = pl.program_id(1)

    @pl.when(s == 0)
    def _():
        o_ref[...] = jnp.zeros_like(o_ref)

    v = cv_ref[s]
    w = cw_ref[s]
    o_ref[pl.ds(v, 1)] += w * x_ref[0][None]


def _plane_pool(batch, chans):
    return pl.pallas_call(
        _body,
        grid_spec=pltpu.PrefetchScalarGridSpec(
            num_scalar_prefetch=3,
            grid=(NCB, NC),
            in_specs=[
                pl.BlockSpec((1, batch, CB),
                             lambda cb, s, ck, cv, cw: (ck[s], 0, cb)),
            ],
            out_specs=pl.BlockSpec((N_OUT, batch, CB),
                                   lambda cb, s, ck, cv, cw: (0, 0, cb)),
        ),
        out_shape=jax.ShapeDtypeStruct((N_OUT, batch, chans), jnp.float32),
    )


def kernel(x, down_neigh_indices):
    b, c, n_in = x.shape
    idx32 = down_neigh_indices.astype(jnp.int32)
    onehot = jax.nn.one_hot(idx32, n_in, axis=-1, dtype=jnp.float32)
    m = onehot.sum(1).T * jnp.float32(1.0 / NEIGH)
    uw, uv = lax.top_k(m, U)
    kk = jnp.broadcast_to(jnp.arange(n_in, dtype=jnp.int32)[:, None],
                          (n_in, U))
    flat_w = uw.reshape(-1)
    flat_v = uv.reshape(-1).astype(jnp.int32)
    flat_k = kk.reshape(-1)
    key = flat_k + jnp.where(flat_w > 0, 0, 10 * n_in * U)
    order = jnp.argsort(key, stable=True)[:NC]
    ck = jnp.where(flat_w[order] > 0, flat_k[order], n_in - 1)
    cv = flat_v[order]
    cw = flat_w[order]
    xt = jnp.transpose(x, (2, 0, 1))
    out_t = _plane_pool(b, c)(ck, cv, cw, xt)
    return jnp.transpose(out_t, (1, 2, 0))


# speedup vs baseline: 1.5483x; 1.5483x over previous
import jax
import jax.numpy as jnp
from jax import lax
from jax.experimental import pallas as pl
from jax.experimental.pallas import tpu as pltpu

N_IN = 162
N_OUT = 42
NEIGH = 7
U = 4
B1, B2, B3 = 60, 137, 158
CB = 512
NCB = 2048 // CB


def _body(ck_ref, uv_ref, uw_ref, x_ref, o_ref):
    s = pl.program_id(1)

    @pl.when(s == 0)
    def _():
        o_ref[...] = jnp.zeros_like(o_ref)

    xb = x_ref[0]

    def rmw(u):
        o_ref[pl.ds(uv_ref[s, u], 1)] += (uw_ref[s, u] * xb)[None]

    rmw(0)

    @pl.when(s >= B1)
    def _():
        rmw(1)

    @pl.when(s >= B2)
    def _():
        rmw(2)

    @pl.when(s >= B3)
    def _():
        rmw(3)


def _plane_pool(batch, chans):
    return pl.pallas_call(
        _body,
        grid_spec=pltpu.PrefetchScalarGridSpec(
            num_scalar_prefetch=3,
            grid=(NCB, N_IN),
            in_specs=[
                pl.BlockSpec((1, batch, CB),
                             lambda cb, s, ck, uv, uw: (ck[s], 0, cb)),
            ],
            out_specs=pl.BlockSpec((N_OUT, batch, CB),
                                   lambda cb, s, ck, uv, uw: (0, 0, cb)),
        ),
        out_shape=jax.ShapeDtypeStruct((N_OUT, batch, chans), jnp.float32),
    )


def kernel(x, down_neigh_indices):
    b, c, n_in = x.shape
    idx32 = down_neigh_indices.astype(jnp.int32)
    onehot = jax.nn.one_hot(idx32, n_in, axis=-1, dtype=jnp.float32)
    m = onehot.sum(1).T * jnp.float32(1.0 / NEIGH)
    uw, uv = lax.top_k(m, U)
    fo = (m > 0).sum(1).astype(jnp.int32)
    order = jnp.argsort(fo * n_in + jnp.arange(n_in, dtype=jnp.int32))
    ck = order.astype(jnp.int32)
    uvs = uv[order].astype(jnp.int32)
    uws = uw[order]
    xt = jnp.transpose(x, (2, 0, 1))
    out_t = _plane_pool(b, c)(ck, uvs, uws, xt)
    return jnp.transpose(out_t, (1, 2, 0))


# speedup vs baseline: 1.5727x; 1.0158x over previous
import jax
import jax.numpy as jnp
from jax import lax
from jax.experimental import pallas as pl
from jax.experimental.pallas import tpu as pltpu

N_IN = 162
N_OUT = 42
NEIGH = 7
U = 4
B1, B2, B3 = 60, 137, 158
CB = 512
NCB = 2048 // CB


def _body(fo_ref, uv_ref, uw_ref, x_ref, o_ref):
    s = pl.program_id(1)

    @pl.when(s == 0)
    def _():
        o_ref[...] = jnp.zeros_like(o_ref)

    xb = x_ref[0]
    fo = fo_ref[s]

    def rmw(u):
        o_ref[pl.ds(uv_ref[s, u], 1)] += (uw_ref[s, u] * xb)[None]

    rmw(0)

    @pl.when(fo > 1)
    def _():
        rmw(1)

    @pl.when(fo > 2)
    def _():
        rmw(2)

    @pl.when(fo > 3)
    def _():
        rmw(3)


def _plane_pool(batch, chans):
    return pl.pallas_call(
        _body,
        grid_spec=pltpu.PrefetchScalarGridSpec(
            num_scalar_prefetch=3,
            grid=(NCB, N_IN),
            in_specs=[
                pl.BlockSpec((1, batch, CB),
                             lambda cb, s, fo, uv, uw: (s, 0, cb)),
            ],
            out_specs=pl.BlockSpec((N_OUT, batch, CB),
                                   lambda cb, s, fo, uv, uw: (0, 0, cb)),
        ),
        out_shape=jax.ShapeDtypeStruct((N_OUT, batch, chans), jnp.float32),
    )


def kernel(x, down_neigh_indices):
    b, c, n_in = x.shape
    idx32 = down_neigh_indices.astype(jnp.int32)
    onehot = jax.nn.one_hot(idx32, n_in, axis=-1, dtype=jnp.float32)
    m = onehot.sum(1).T * jnp.float32(1.0 / NEIGH)
    uw, uv = lax.top_k(m, U)
    fo = (m > 0).sum(1).astype(jnp.int32)
    xt = jnp.transpose(x, (2, 0, 1))
    out_t = _plane_pool(b, c)(fo, uv.astype(jnp.int32), uw, xt)
    return jnp.transpose(out_t, (1, 2, 0))


# speedup vs baseline: 2.5806x; 1.6409x over previous
import jax
import jax.numpy as jnp
from jax import lax
from jax.experimental import pallas as pl
from jax.experimental.pallas import tpu as pltpu

N_IN = 162
N_OUT = 42
NEIGH = 7
U = 4
B1, B2, B3 = 60, 137, 158
CB = 1024
NCB = 2048 // CB


def _body(fo_ref, uv_ref, uw_ref, x_ref, o_ref):
    s = pl.program_id(1)

    @pl.when(s == 0)
    def _():
        o_ref[...] = jnp.zeros_like(o_ref)

    xb = x_ref[0]
    fo = fo_ref[s]

    def rmw(u):
        o_ref[pl.ds(uv_ref[s, u], 1)] += (uw_ref[s, u] * xb)[None]

    rmw(0)

    @pl.when(fo > 1)
    def _():
        rmw(1)

    @pl.when(fo > 2)
    def _():
        rmw(2)

    @pl.when(fo > 3)
    def _():
        rmw(3)


def _plane_pool(batch, chans):
    return pl.pallas_call(
        _body,
        grid_spec=pltpu.PrefetchScalarGridSpec(
            num_scalar_prefetch=3,
            grid=(NCB, N_IN),
            in_specs=[
                pl.BlockSpec((1, batch, CB),
                             lambda cb, s, fo, uv, uw: (s, 0, cb)),
            ],
            out_specs=pl.BlockSpec((N_OUT, batch, CB),
                                   lambda cb, s, fo, uv, uw: (0, 0, cb)),
        ),
        out_shape=jax.ShapeDtypeStruct((N_OUT, batch, chans), jnp.float32),
    )


def kernel(x, down_neigh_indices):
    b, c, n_in = x.shape
    idx32 = down_neigh_indices.astype(jnp.int32)
    onehot = jax.nn.one_hot(idx32, n_in, axis=-1, dtype=jnp.float32)
    m = onehot.sum(1).T * jnp.float32(1.0 / NEIGH)
    uw, uv = lax.top_k(m, U)
    fo = (m > 0).sum(1).astype(jnp.int32)
    xt = jnp.transpose(x, (2, 0, 1))
    out_t = _plane_pool(b, c)(fo, uv.astype(jnp.int32), uw, xt)
    return jnp.transpose(out_t, (1, 2, 0))


# speedup vs baseline: 3.8733x; 1.5009x over previous
import jax
import jax.numpy as jnp
from jax import lax
from jax.experimental import pallas as pl
from jax.experimental.pallas import tpu as pltpu

N_IN = 162
N_OUT = 42
NEIGH = 7
U = 4
B1, B2, B3 = 60, 137, 158
CB = 2048
NCB = 2048 // CB


def _body(fo_ref, uv_ref, uw_ref, x_ref, o_ref):
    s = pl.program_id(1)

    @pl.when(s == 0)
    def _():
        o_ref[...] = jnp.zeros_like(o_ref)

    xb = x_ref[0]
    fo = fo_ref[s]

    def rmw(u):
        o_ref[pl.ds(uv_ref[s, u], 1)] += (uw_ref[s, u] * xb)[None]

    rmw(0)

    @pl.when(fo > 1)
    def _():
        rmw(1)

    @pl.when(fo > 2)
    def _():
        rmw(2)

    @pl.when(fo > 3)
    def _():
        rmw(3)


def _plane_pool(batch, chans):
    return pl.pallas_call(
        _body,
        grid_spec=pltpu.PrefetchScalarGridSpec(
            num_scalar_prefetch=3,
            grid=(NCB, N_IN),
            in_specs=[
                pl.BlockSpec((1, batch, CB),
                             lambda cb, s, fo, uv, uw: (s, 0, cb)),
            ],
            out_specs=pl.BlockSpec((N_OUT, batch, CB),
                                   lambda cb, s, fo, uv, uw: (0, 0, cb)),
        ),
        out_shape=jax.ShapeDtypeStruct((N_OUT, batch, chans), jnp.float32),
    )


def kernel(x, down_neigh_indices):
    b, c, n_in = x.shape
    idx32 = down_neigh_indices.astype(jnp.int32)
    onehot = jax.nn.one_hot(idx32, n_in, axis=-1, dtype=jnp.float32)
    m = onehot.sum(1).T * jnp.float32(1.0 / NEIGH)
    uw, uv = lax.top_k(m, U)
    fo = (m > 0).sum(1).astype(jnp.int32)
    xt = jnp.transpose(x, (2, 0, 1))
    out_t = _plane_pool(b, c)(fo, uv.astype(jnp.int32), uw, xt)
    return jnp.transpose(out_t, (1, 2, 0))
